# baseline (speedup 1.0000x reference)
import functools

import jax
import jax.numpy as jnp
from jax import lax
from jax.experimental import pallas as pl
from jax.experimental.pallas import tpu as pltpu
from jax.experimental.pallas import tpu_sc as plsc

CUTOFF2 = 25.0
N_NODES = 100000
N_EDGES = 6400000
N_MOL = 1024
MAX_Z = 100

NC, NS, LANES = 2, 16, 16
NW = NC * NS
NODES_PAD = 100352
NODES_PER_W = NODES_PAD // NW
EDGES_PER_W = N_EDGES // NW
CHUNK = 800
NCHUNK = EDGES_PER_W // CHUNK
RSCALE = float(2 ** 20)

_mesh = plsc.VectorSubcoreMesh(core_axis_name="c", subcore_axis_name="s")
_sc_params = pltpu.CompilerParams(needs_layout_passes=False)


def _table_body(z_hbm, film_hbm, idxm_hbm, rtab_hbm, packed_hbm,
                z_v, film_v, idxm_v, rtab_v, packed_v):
    wid = lax.axis_index("s") * NC + lax.axis_index("c")
    base = wid * NODES_PER_W
    pltpu.sync_copy(rtab_hbm, rtab_v)
    pltpu.sync_copy(z_hbm.at[pl.ds(base, NODES_PER_W)], z_v)
    pltpu.sync_copy(film_hbm.at[pl.ds(base, NODES_PER_W)], film_v)
    pltpu.sync_copy(idxm_hbm.at[pl.ds(base, NODES_PER_W)], idxm_v)

    @pl.loop(0, NODES_PER_W // LANES)
    def _node_vec(t):
        o = t * LANES
        z = z_v[pl.ds(o, LANES)]
        f = film_v[pl.ds(o, LANES)]
        m = idxm_v[pl.ds(o, LANES)]
        r = plsc.load_gather(rtab_v, [f * MAX_Z + z])
        u = (r * RSCALE + 0.5).astype(jnp.int32)
        packed_v[pl.ds(o, LANES)] = (u << 10) | m

    pltpu.sync_copy(packed_v, packed_hbm.at[pl.ds(base, NODES_PER_W)])


_build_table = pl.kernel(
    _table_body,
    out_type=jax.ShapeDtypeStruct((NODES_PAD,), jnp.int32),
    mesh=_mesh,
    scratch_types=[
        pltpu.VMEM((NODES_PER_W,), jnp.int32),
        pltpu.VMEM((NODES_PER_W,), jnp.int32),
        pltpu.VMEM((NODES_PER_W,), jnp.int32),
        pltpu.VMEM((2 * MAX_Z,), jnp.float32),
        pltpu.VMEM((NODES_PER_W,), jnp.int32),
    ],
    compiler_params=_sc_params,
)


def _edge_body(packed_hbm, idx_i_hbm, idx_j_hbm, x_hbm, y_hbm, z_hbm, out_hbm,
               table_v, acc_v, ii0, jj0, x0, y0, z0, ii1, jj1, x1, y1, z1,
               row_v, sem0, sem1):
    wid = lax.axis_index("s") * NC + lax.axis_index("c")
    ebase = wid * EDGES_PER_W
    pltpu.sync_copy(packed_hbm, table_v)

    zeros = jnp.zeros((LANES,), jnp.float32)

    @pl.loop(0, N_MOL // LANES)
    def _zero(c):
        o = c * LANES
        for l in range(LANES):
            acc_v[l, pl.ds(o, LANES)] = zeros

    lane = lax.iota(jnp.int32, LANES)
    bufs = ((ii0, jj0, x0, y0, z0, sem0), (ii1, jj1, x1, y1, z1, sem1))

    def issue(buf, k):
        ii_v, jj_v, x_v, y_v, z_v, sem = buf
        cb = ebase + k * CHUNK
        pltpu.async_copy(idx_i_hbm.at[pl.ds(cb, CHUNK)], ii_v, sem)
        pltpu.async_copy(idx_j_hbm.at[pl.ds(cb, CHUNK)], jj_v, sem)
        pltpu.async_copy(x_hbm.at[pl.ds(cb, CHUNK)], x_v, sem)
        pltpu.async_copy(y_hbm.at[pl.ds(cb, CHUNK)], y_v, sem)
        pltpu.async_copy(z_hbm.at[pl.ds(cb, CHUNK)], z_v, sem)

    def drain(buf):
        ii_v, jj_v, x_v, y_v, z_v, sem = buf
        pltpu.make_async_copy(idx_i_hbm.at[pl.ds(0, CHUNK)], ii_v, sem).wait()
        pltpu.make_async_copy(idx_j_hbm.at[pl.ds(0, CHUNK)], jj_v, sem).wait()
        pltpu.make_async_copy(x_hbm.at[pl.ds(0, CHUNK)], x_v, sem).wait()
        pltpu.make_async_copy(y_hbm.at[pl.ds(0, CHUNK)], y_v, sem).wait()
        pltpu.make_async_copy(z_hbm.at[pl.ds(0, CHUNK)], z_v, sem).wait()

    def process(buf):
        ii_v, jj_v, x_v, y_v, z_v, _ = buf

        @plsc.parallel_loop(0, CHUNK // LANES, unroll=8)
        def _vec(t):
            o = t * LANES
            ii = ii_v[pl.ds(o, LANES)]
            jj = jj_v[pl.ds(o, LANES)]
            x = x_v[pl.ds(o, LANES)]
            y = y_v[pl.ds(o, LANES)]
            z = z_v[pl.ds(o, LANES)]
            pi = plsc.load_gather(table_v, [ii])
            pj = plsc.load_gather(table_v, [jj])
            mol = pi & (N_MOL - 1)
            ri = (pi >> 10).astype(jnp.float32)
            rj = (pj >> 10).astype(jnp.float32)
            s = (ri + rj) * (1.0 / RSCALE)
            d2 = x * x + y * y + z * z
            s2 = s * s
            num = s2 * s2 * s2
            den = d2 * d2 * d2
            pot = jnp.where(d2 <= CUTOFF2, num / den, 0.0)
            plsc.addupdate_scatter(acc_v, [lane, mol], pot)

    issue(bufs[0], 0)

    @pl.loop(0, NCHUNK, step=2)
    def _chunk(k):
        issue(bufs[1], k + 1)
        drain(bufs[0])
        process(bufs[0])

        @pl.when(k + 2 < NCHUNK)
        def _():
            issue(bufs[0], k + 2)

        drain(bufs[1])
        process(bufs[1])

    @pl.loop(0, N_MOL // LANES)
    def _fold(c):
        o = c * LANES
        acc = acc_v[0, pl.ds(o, LANES)]
        for l in range(1, LANES):
            acc = acc + acc_v[l, pl.ds(o, LANES)]
        row_v[pl.ds(o, LANES)] = acc

    pltpu.sync_copy(row_v, out_hbm.at[wid])


_edge_kernel = pl.kernel(
    _edge_body,
    out_type=jax.ShapeDtypeStruct((NW, N_MOL), jnp.float32),
    mesh=_mesh,
    scratch_types=[
        pltpu.VMEM((NODES_PAD,), jnp.int32),
        pltpu.VMEM((LANES, N_MOL), jnp.float32),
        pltpu.VMEM((CHUNK,), jnp.int32),
        pltpu.VMEM((CHUNK,), jnp.int32),
        pltpu.VMEM((CHUNK,), jnp.float32),
        pltpu.VMEM((CHUNK,), jnp.float32),
        pltpu.VMEM((CHUNK,), jnp.float32),
        pltpu.VMEM((CHUNK,), jnp.int32),
        pltpu.VMEM((CHUNK,), jnp.int32),
        pltpu.VMEM((CHUNK,), jnp.float32),
        pltpu.VMEM((CHUNK,), jnp.float32),
        pltpu.VMEM((CHUNK,), jnp.float32),
        pltpu.VMEM((N_MOL,), jnp.float32),
        pltpu.SemaphoreType.DMA,
        pltpu.SemaphoreType.DMA,
    ],
    compiler_params=_sc_params,
)


def _reduce_body(p_ref, o_ref):
    o_ref[...] = jnp.sum(p_ref[...], axis=0)


_reduce = pl.pallas_call(
    _reduce_body,
    out_shape=jax.ShapeDtypeStruct((N_MOL,), jnp.float32),
)


def kernel(Z, idx_m, Rij, idx_i, idx_j, is_film, radius_table):
    pad = NODES_PAD - N_NODES
    z_p = jnp.pad(Z.astype(jnp.int32), (0, pad))
    f_p = jnp.pad(is_film.astype(jnp.int32), (0, pad))
    m_p = jnp.pad(idx_m.astype(jnp.int32), (0, pad))
    rtab = radius_table.reshape(-1).astype(jnp.float32)
    packed = _build_table(z_p, f_p, m_p, rtab)
    rt = Rij.T
    partials = _edge_kernel(packed, idx_i.astype(jnp.int32),
                            idx_j.astype(jnp.int32), rt[0], rt[1], rt[2])
    return _reduce(partials)


# speedup vs baseline: 1.3951x; 1.3951x over previous
import functools

import jax
import jax.numpy as jnp
from jax import lax
from jax.experimental import pallas as pl
from jax.experimental.pallas import tpu as pltpu
from jax.experimental.pallas import tpu_sc as plsc

CUTOFF2 = 25.0
N_NODES = 100000
N_EDGES = 6400000
N_MOL = 1024
MAX_Z = 100

NC, NS, LANES = 2, 16, 16
NW = NC * NS
NODES_PAD = 100352
NODES_PER_W = NODES_PAD // NW
EDGES_PER_W = N_EDGES // NW
CHUNK = 2000
NCHUNK = EDGES_PER_W // CHUNK
D2B = 51200
RSCALE = float(2 ** 20)

_mesh = plsc.VectorSubcoreMesh(core_axis_name="c", subcore_axis_name="s")
_sc_params = pltpu.CompilerParams(needs_layout_passes=False)


def _table_body(z_hbm, film_hbm, idxm_hbm, rtab_hbm, packed_hbm,
                z_v, film_v, idxm_v, rtab_v, packed_v):
    wid = lax.axis_index("s") * NC + lax.axis_index("c")
    base = wid * NODES_PER_W
    pltpu.sync_copy(rtab_hbm, rtab_v)
    pltpu.sync_copy(z_hbm.at[pl.ds(base, NODES_PER_W)], z_v)
    pltpu.sync_copy(film_hbm.at[pl.ds(base, NODES_PER_W)], film_v)
    pltpu.sync_copy(idxm_hbm.at[pl.ds(base, NODES_PER_W)], idxm_v)

    @pl.loop(0, NODES_PER_W // LANES)
    def _node_vec(t):
        o = t * LANES
        z = z_v[pl.ds(o, LANES)]
        f = film_v[pl.ds(o, LANES)]
        m = idxm_v[pl.ds(o, LANES)]
        r = plsc.load_gather(rtab_v, [f * MAX_Z + z])
        u = (r * RSCALE + 0.5).astype(jnp.int32)
        packed_v[pl.ds(o, LANES)] = (u << 10) | m

    pltpu.sync_copy(packed_v, packed_hbm.at[pl.ds(base, NODES_PER_W)])


_build_table = pl.kernel(
    _table_body,
    out_type=jax.ShapeDtypeStruct((NODES_PAD,), jnp.int32),
    mesh=_mesh,
    scratch_types=[
        pltpu.VMEM((NODES_PER_W,), jnp.int32),
        pltpu.VMEM((NODES_PER_W,), jnp.int32),
        pltpu.VMEM((NODES_PER_W,), jnp.int32),
        pltpu.VMEM((2 * MAX_Z,), jnp.float32),
        pltpu.VMEM((NODES_PER_W,), jnp.int32),
    ],
    compiler_params=_sc_params,
)


def _edge_body(packed_hbm, idx_i_hbm, idx_j_hbm, d2_hbm, out_hbm,
               table_v, acc_v, ii0, jj0, d0, ii1, jj1, d1,
               row_v, sem0, sem1):
    wid = lax.axis_index("s") * NC + lax.axis_index("c")
    ebase = wid * EDGES_PER_W
    pltpu.sync_copy(packed_hbm, table_v)

    zeros = jnp.zeros((LANES,), jnp.float32)

    @pl.loop(0, N_MOL // LANES)
    def _zero(c):
        o = c * LANES
        for l in range(LANES):
            acc_v[l, pl.ds(o, LANES)] = zeros

    lane = lax.iota(jnp.int32, LANES)
    bufs = ((ii0, jj0, d0, sem0), (ii1, jj1, d1, sem1))

    def issue(buf, k):
        ii_v, jj_v, d_v, sem = buf
        cb = ebase + k * CHUNK
        pltpu.async_copy(idx_i_hbm.at[pl.ds(cb, CHUNK)], ii_v, sem)
        pltpu.async_copy(idx_j_hbm.at[pl.ds(cb, CHUNK)], jj_v, sem)
        pltpu.async_copy(d2_hbm.at[pl.ds(cb, CHUNK)], d_v, sem)

    def drain(buf):
        ii_v, jj_v, d_v, sem = buf
        pltpu.make_async_copy(idx_i_hbm.at[pl.ds(0, CHUNK)], ii_v, sem).wait()
        pltpu.make_async_copy(idx_j_hbm.at[pl.ds(0, CHUNK)], jj_v, sem).wait()
        pltpu.make_async_copy(d2_hbm.at[pl.ds(0, CHUNK)], d_v, sem).wait()

    def process(buf):
        ii_v, jj_v, d_v, _ = buf

        @plsc.parallel_loop(0, CHUNK // LANES, unroll=8)
        def _vec(t):
            o = t * LANES
            ii = ii_v[pl.ds(o, LANES)]
            jj = jj_v[pl.ds(o, LANES)]
            d2 = d_v[pl.ds(o, LANES)]
            pi = plsc.load_gather(table_v, [ii])
            pj = plsc.load_gather(table_v, [jj])
            mol = pi & (N_MOL - 1)
            ri = (pi >> 10).astype(jnp.float32)
            rj = (pj >> 10).astype(jnp.float32)
            s = (ri + rj) * (1.0 / RSCALE)
            s2 = s * s
            num = s2 * s2 * s2
            den = d2 * d2 * d2
            pot = jnp.where(d2 <= CUTOFF2, num / den, 0.0)
            plsc.addupdate_scatter(acc_v, [lane, mol], pot)

    issue(bufs[0], 0)

    @pl.loop(0, NCHUNK, step=2)
    def _chunk(k):
        issue(bufs[1], k + 1)
        drain(bufs[0])
        process(bufs[0])

        @pl.when(k + 2 < NCHUNK)
        def _():
            issue(bufs[0], k + 2)

        drain(bufs[1])
        process(bufs[1])

    @pl.loop(0, N_MOL // LANES)
    def _fold(c):
        o = c * LANES
        acc = acc_v[0, pl.ds(o, LANES)]
        for l in range(1, LANES):
            acc = acc + acc_v[l, pl.ds(o, LANES)]
        row_v[pl.ds(o, LANES)] = acc

    pltpu.sync_copy(row_v, out_hbm.at[wid])


_edge_kernel = pl.kernel(
    _edge_body,
    out_type=jax.ShapeDtypeStruct((NW, N_MOL), jnp.float32),
    mesh=_mesh,
    scratch_types=[
        pltpu.VMEM((NODES_PAD,), jnp.int32),
        pltpu.VMEM((LANES, N_MOL), jnp.float32),
        pltpu.VMEM((CHUNK,), jnp.int32),
        pltpu.VMEM((CHUNK,), jnp.int32),
        pltpu.VMEM((CHUNK,), jnp.float32),
        pltpu.VMEM((CHUNK,), jnp.int32),
        pltpu.VMEM((CHUNK,), jnp.int32),
        pltpu.VMEM((CHUNK,), jnp.float32),
        pltpu.VMEM((N_MOL,), jnp.float32),
        pltpu.SemaphoreType.DMA,
        pltpu.SemaphoreType.DMA,
    ],
    compiler_params=_sc_params,
)


def _d2_body(r_ref, o_ref):
    r = r_ref[...]
    sq = r * r
    o_ref[...] = sq[0] + sq[1] + sq[2]


_d2 = pl.pallas_call(
    _d2_body,
    grid=(N_EDGES // D2B,),
    in_specs=[pl.BlockSpec((3, D2B), lambda i: (0, i))],
    out_specs=pl.BlockSpec((D2B,), lambda i: (i,)),
    out_shape=jax.ShapeDtypeStruct((N_EDGES,), jnp.float32),
)


def _reduce_body(p_ref, o_ref):
    o_ref[...] = jnp.sum(p_ref[...], axis=0)


_reduce = pl.pallas_call(
    _reduce_body,
    out_shape=jax.ShapeDtypeStruct((N_MOL,), jnp.float32),
)


def kernel(Z, idx_m, Rij, idx_i, idx_j, is_film, radius_table):
    pad = NODES_PAD - N_NODES
    z_p = jnp.pad(Z.astype(jnp.int32), (0, pad))
    f_p = jnp.pad(is_film.astype(jnp.int32), (0, pad))
    m_p = jnp.pad(idx_m.astype(jnp.int32), (0, pad))
    rtab = radius_table.reshape(-1).astype(jnp.float32)
    packed = _build_table(z_p, f_p, m_p, rtab)
    d2 = _d2(Rij.T)
    partials = _edge_kernel(packed, idx_i.astype(jnp.int32),
                            idx_j.astype(jnp.int32), d2)
    return _reduce(partials)
